# Initial kernel scaffold; baseline (speedup 1.0000x reference)
import jax
import jax.numpy as jnp
from jax.experimental import pallas as pl


def kernel(z_a, z_b, z_a_local_features, z_b_local_features, grid_a, grid_b):
    raise NotImplementedError("write your pallas kernel here")


# speedup vs baseline: 1.0401x; 1.0401x over previous
import functools

import jax
import jax.numpy as jnp
from jax import lax
from jax.experimental import pallas as pl

B, N, C, D = 16, 1024, 384, 2048
K_A, K_B = 20, 4
LAMBDA_PARAM = 25.0
ALPHA = 0.25
EPS = 1e-4

_DOT = dict(preferred_element_type=jnp.float32,
            precision=jax.lax.Precision.HIGHEST)


def _vicreg_global(za, zb):
    inv_g = jnp.sum((za - zb) ** 2, keepdims=True) / (B * D)

    def half(x):
        mu = jnp.mean(x, axis=0, keepdims=True)
        xc = x - mu
        ss = jnp.sum(xc * xc, axis=0, keepdims=True)
        std = jnp.sqrt(ss / (B - 1) + EPS)
        var_l = jnp.sum(jnp.maximum(1.0 - std, 0.0), keepdims=True) / D
        gram = lax.dot_general(xc, xc, (((1,), (1,)), ((), ())), **_DOT)
        fro2 = jnp.sum(gram * gram, keepdims=True)
        diag2 = jnp.sum(ss * ss, keepdims=True)
        cov_l = (fro2 - diag2) / ((B - 1) * (B - 1) * D)
        return var_l, cov_l

    va, ca = half(za)
    vb, cb = half(zb)
    return 25.0 * inv_g + 12.5 * (va + vb) + (ca + cb)


def _kern(a_ref, b_ref, gax_ref, gay_ref, gbx_ref, gby_ref,
          za_ref, zb_ref, out_ref):
    bi = pl.program_id(0)

    @pl.when(bi == 0)
    def _():
        out_ref[...] = ALPHA * _vicreg_global(za_ref[...], zb_ref[...])

    a = a_ref[0]
    bm = b_ref[0]
    ones_r = jnp.ones((1, C), jnp.float32)
    a2 = lax.dot_general(a * a, ones_r, (((1,), (1,)), ((), ())), **_DOT)
    b2 = lax.dot_general(ones_r, bm * bm, (((1,), (1,)), ((), ())), **_DOT)
    f = lax.dot_general(a, bm, (((1,), (1,)), ((), ())), **_DOT)
    d2f = jnp.maximum(a2 + b2 - 2.0 * f, 0.0)

    rmin_f = jnp.min(d2f, axis=1, keepdims=True)
    cmin_f = jnp.min(d2f, axis=0, keepdims=True)

    gax, gay = gax_ref[0], gay_ref[0]
    gbx, gby = gbx_ref[0], gby_ref[0]
    ga2 = gax * gax + gay * gay
    gb2 = gbx * gbx + gby * gby
    g = jnp.sqrt(jnp.maximum(ga2 + gb2 - 2.0 * (gax * gbx + gay * gby), 0.0))

    colidx = lax.broadcasted_iota(jnp.int32, (N, N), 1)
    rowidx = lax.broadcasted_iota(jnp.int32, (N, N), 0)
    big = jnp.int32(2**30)

    gm_a = jnp.min(g, axis=1, keepdims=True)
    ja = jnp.min(jnp.where(g == gm_a, colidx, big), axis=1, keepdims=True)
    fsel_a = jnp.sum(jnp.where(colidx == ja, d2f, 0.0), axis=1, keepdims=True)

    gm_b = jnp.min(g, axis=0, keepdims=True)
    ib = jnp.min(jnp.where(g == gm_b, rowidx, big), axis=0, keepdims=True)
    fsel_b = jnp.sum(jnp.where(rowidx == ib, d2f, 0.0), axis=0, keepdims=True)

    kc = jnp.concatenate([rmin_f, gm_a], axis=1)
    pc = jnp.concatenate([rmin_f, fsel_a], axis=1)
    ridx2 = lax.broadcasted_iota(jnp.int32, (N, 2), 0)
    big = jnp.int32(2**30)

    def body_c(r, carry):
        kcur, acc = carry
        m = jnp.min(kcur, axis=0, keepdims=True)
        sel_i = jnp.min(jnp.where(kcur == m, ridx2, big),
                        axis=0, keepdims=True)
        sel = ridx2 == sel_i
        acc = acc + jnp.sum(jnp.where(sel, pc, 0.0), axis=0, keepdims=True)
        kcur = jnp.where(sel, jnp.inf, kcur)
        return kcur, acc

    _, acc_c = lax.fori_loop(0, K_A, body_c,
                             (kc, jnp.zeros((1, 2), jnp.float32)))

    kr = jnp.concatenate([cmin_f, gm_b], axis=0)
    pr = jnp.concatenate([cmin_f, fsel_b], axis=0)
    cidx2 = lax.broadcasted_iota(jnp.int32, (2, N), 1)
    klim = jnp.where(lax.broadcasted_iota(jnp.int32, (2, 1), 0) == 0,
                     float(K_A), float(K_B))

    def body_r(r, carry):
        kcur, acc = carry
        m = jnp.min(kcur, axis=1, keepdims=True)
        sel_j = jnp.min(jnp.where(kcur == m, cidx2, big),
                        axis=1, keepdims=True)
        sel = cidx2 == sel_j
        pay = jnp.sum(jnp.where(sel, pr, 0.0), axis=1, keepdims=True)
        w = (klim > r.astype(jnp.float32)).astype(jnp.float32)
        acc = acc + pay * w
        kcur = jnp.where(sel, jnp.inf, kcur)
        return kcur, acc

    _, acc_r = lax.fori_loop(0, K_A, body_r,
                             (kr, jnp.zeros((2, 1), jnp.float32)))

    s_feat_a = acc_c[0:1, 0:1]
    s_grid_a = acc_c[0:1, 1:2]
    s_feat_b = acc_r[0:1, 0:1]
    s_grid_b = acc_r[1:2, 0:1]

    c20 = (1.0 - ALPHA) * LAMBDA_PARAM / (2.0 * B * K_A * C)
    c4 = (1.0 - ALPHA) * LAMBDA_PARAM / (2.0 * B * K_B * C)
    out_ref[...] += (s_feat_a + s_feat_b + s_grid_a) * c20 + s_grid_b * c4


@functools.partial(jax.jit, static_argnames=())
def kernel(z_a, z_b, z_a_local_features, z_b_local_features, grid_a, grid_b):
    a = z_a_local_features.reshape(B, N, C)
    bm = z_b_local_features.reshape(B, N, C)
    ga = grid_a.reshape(B, N, 2)
    gb = grid_b.reshape(B, N, 2)
    gax = ga[..., 0:1]
    gay = ga[..., 1:2]
    gbx = gb[..., 0][:, None, :]
    gby = gb[..., 1][:, None, :]

    out = pl.pallas_call(
        _kern,
        grid=(B,),
        in_specs=[
            pl.BlockSpec((1, N, C), lambda i: (i, 0, 0)),
            pl.BlockSpec((1, N, C), lambda i: (i, 0, 0)),
            pl.BlockSpec((1, N, 1), lambda i: (i, 0, 0)),
            pl.BlockSpec((1, N, 1), lambda i: (i, 0, 0)),
            pl.BlockSpec((1, 1, N), lambda i: (i, 0, 0)),
            pl.BlockSpec((1, 1, N), lambda i: (i, 0, 0)),
            pl.BlockSpec((B, D), lambda i: (0, 0)),
            pl.BlockSpec((B, D), lambda i: (0, 0)),
        ],
        out_specs=pl.BlockSpec((1, 1), lambda i: (0, 0)),
        out_shape=jax.ShapeDtypeStruct((1, 1), jnp.float32),
    )(a, bm, gax, gay, gbx, gby, z_a, z_b)
    return out[0, 0]


# speedup vs baseline: 1.2888x; 1.2392x over previous
import functools

import jax
import jax.numpy as jnp
from jax import lax
from jax.experimental import pallas as pl

B, N, C, D = 16, 1024, 384, 2048
K_A, K_B = 20, 4
LAMBDA_PARAM = 25.0
ALPHA = 0.25
EPS = 1e-4

_DOT = dict(preferred_element_type=jnp.float32,
            precision=jax.lax.Precision.HIGHEST)


def _vicreg_global(za, zb):
    inv_g = jnp.sum((za - zb) ** 2, keepdims=True) / (B * D)

    def half(x):
        mu = jnp.mean(x, axis=0, keepdims=True)
        xc = x - mu
        ss = jnp.sum(xc * xc, axis=0, keepdims=True)
        std = jnp.sqrt(ss / (B - 1) + EPS)
        var_l = jnp.sum(jnp.maximum(1.0 - std, 0.0), keepdims=True) / D
        gram = lax.dot_general(xc, xc, (((1,), (1,)), ((), ())), **_DOT)
        fro2 = jnp.sum(gram * gram, keepdims=True)
        diag2 = jnp.sum(ss * ss, keepdims=True)
        cov_l = (fro2 - diag2) / ((B - 1) * (B - 1) * D)
        return var_l, cov_l

    va, ca = half(za)
    vb, cb = half(zb)
    return 25.0 * inv_g + 12.5 * (va + vb) + (ca + cb)


def _kern(a_ref, b_ref, gax_ref, gay_ref, gbx_ref, gby_ref,
          za_ref, zb_ref, out_ref):
    bi = pl.program_id(0)

    @pl.when(bi == 0)
    def _():
        out_ref[...] = ALPHA * _vicreg_global(za_ref[...], zb_ref[...])

    a = a_ref[0]
    bm = b_ref[0]
    ones_r = jnp.ones((1, C), jnp.float32)
    a2 = lax.dot_general(a * a, ones_r, (((1,), (1,)), ((), ())), **_DOT)
    b2 = lax.dot_general(ones_r, bm * bm, (((1,), (1,)), ((), ())), **_DOT)
    f = lax.dot_general(a, bm, (((1,), (1,)), ((), ())),
                        preferred_element_type=jnp.float32)
    d2f = jnp.maximum(a2 + b2 - 2.0 * f, 0.0)

    rmin_f = jnp.min(d2f, axis=1, keepdims=True)
    cmin_f = jnp.min(d2f, axis=0, keepdims=True)

    gax, gay = gax_ref[0], gay_ref[0]
    gbx, gby = gbx_ref[0], gby_ref[0]
    ga2 = gax * gax + gay * gay
    gb2 = gbx * gbx + gby * gby
    g = jnp.sqrt(jnp.maximum(ga2 + gb2 - 2.0 * (gax * gbx + gay * gby), 0.0))

    colidx = lax.broadcasted_iota(jnp.int32, (N, N), 1)
    rowidx = lax.broadcasted_iota(jnp.int32, (N, N), 0)
    big = jnp.int32(2**30)

    gm_a = jnp.min(g, axis=1, keepdims=True)
    ja = jnp.min(jnp.where(g == gm_a, colidx, big), axis=1, keepdims=True)
    fsel_a = jnp.sum(jnp.where(colidx == ja, d2f, 0.0), axis=1, keepdims=True)

    gm_b = jnp.min(g, axis=0, keepdims=True)
    ib = jnp.min(jnp.where(g == gm_b, rowidx, big), axis=0, keepdims=True)
    fsel_b = jnp.sum(jnp.where(rowidx == ib, d2f, 0.0), axis=0, keepdims=True)

    kc = jnp.concatenate([rmin_f, gm_a], axis=1)
    pc = jnp.concatenate([rmin_f, fsel_a], axis=1)
    ridx2 = lax.broadcasted_iota(jnp.int32, (N, 2), 0)
    big = jnp.int32(2**30)

    def body_c(r, carry):
        kcur, acc = carry
        m = jnp.min(kcur, axis=0, keepdims=True)
        sel_i = jnp.min(jnp.where(kcur == m, ridx2, big),
                        axis=0, keepdims=True)
        sel = ridx2 == sel_i
        acc = acc + jnp.sum(jnp.where(sel, pc, 0.0), axis=0, keepdims=True)
        kcur = jnp.where(sel, jnp.inf, kcur)
        return kcur, acc

    _, acc_c = lax.fori_loop(0, K_A, body_c,
                             (kc, jnp.zeros((1, 2), jnp.float32)))

    kr = jnp.concatenate([cmin_f, gm_b], axis=0)
    pr = jnp.concatenate([cmin_f, fsel_b], axis=0)
    cidx2 = lax.broadcasted_iota(jnp.int32, (2, N), 1)
    klim = jnp.where(lax.broadcasted_iota(jnp.int32, (2, 1), 0) == 0,
                     float(K_A), float(K_B))

    def body_r(r, carry):
        kcur, acc = carry
        m = jnp.min(kcur, axis=1, keepdims=True)
        sel_j = jnp.min(jnp.where(kcur == m, cidx2, big),
                        axis=1, keepdims=True)
        sel = cidx2 == sel_j
        pay = jnp.sum(jnp.where(sel, pr, 0.0), axis=1, keepdims=True)
        w = (klim > r.astype(jnp.float32)).astype(jnp.float32)
        acc = acc + pay * w
        kcur = jnp.where(sel, jnp.inf, kcur)
        return kcur, acc

    _, acc_r = lax.fori_loop(0, K_A, body_r,
                             (kr, jnp.zeros((2, 1), jnp.float32)))

    s_feat_a = acc_c[0:1, 0:1]
    s_grid_a = acc_c[0:1, 1:2]
    s_feat_b = acc_r[0:1, 0:1]
    s_grid_b = acc_r[1:2, 0:1]

    c20 = (1.0 - ALPHA) * LAMBDA_PARAM / (2.0 * B * K_A * C)
    c4 = (1.0 - ALPHA) * LAMBDA_PARAM / (2.0 * B * K_B * C)
    out_ref[...] += (s_feat_a + s_feat_b + s_grid_a) * c20 + s_grid_b * c4


@functools.partial(jax.jit, static_argnames=())
def kernel(z_a, z_b, z_a_local_features, z_b_local_features, grid_a, grid_b):
    a = z_a_local_features.reshape(B, N, C)
    bm = z_b_local_features.reshape(B, N, C)
    ga = grid_a.reshape(B, N, 2)
    gb = grid_b.reshape(B, N, 2)
    gax = ga[..., 0:1]
    gay = ga[..., 1:2]
    gbx = gb[..., 0][:, None, :]
    gby = gb[..., 1][:, None, :]

    out = pl.pallas_call(
        _kern,
        grid=(B,),
        in_specs=[
            pl.BlockSpec((1, N, C), lambda i: (i, 0, 0)),
            pl.BlockSpec((1, N, C), lambda i: (i, 0, 0)),
            pl.BlockSpec((1, N, 1), lambda i: (i, 0, 0)),
            pl.BlockSpec((1, N, 1), lambda i: (i, 0, 0)),
            pl.BlockSpec((1, 1, N), lambda i: (i, 0, 0)),
            pl.BlockSpec((1, 1, N), lambda i: (i, 0, 0)),
            pl.BlockSpec((B, D), lambda i: (0, 0)),
            pl.BlockSpec((B, D), lambda i: (0, 0)),
        ],
        out_specs=pl.BlockSpec((1, 1), lambda i: (0, 0)),
        out_shape=jax.ShapeDtypeStruct((1, 1), jnp.float32),
    )(a, bm, gax, gay, gbx, gby, z_a, z_b)
    return out[0, 0]
